# baseline (speedup 1.0000x reference)
import jax
import jax.numpy as jnp
from jax.experimental import pallas as pl
from jax.experimental.pallas import tpu as pltpu

N = 10000
D = 256
H = 64
C = 3
CP = 8
RB = 200


def _xw_kernel(x_ref, w1_ref, o_ref):
    o_ref[...] = jnp.dot(x_ref[...], w1_ref[...],
                         preferred_element_type=jnp.float32)


def _layer1_kernel(a_ref, xw_ref, b1_ref, w2_ref, hw_ref):
    acc = jnp.dot(a_ref[...], xw_ref[...], preferred_element_type=jnp.float32)
    h = jnp.maximum(acc + b1_ref[...], 0.0)
    hw_ref[...] = jnp.dot(h, w2_ref[...], preferred_element_type=jnp.float32)


def _layer2_kernel(a_ref, hw_ref, b2_ref, out_ref):
    logits = jnp.dot(a_ref[...], hw_ref[...],
                     preferred_element_type=jnp.float32) + b2_ref[...]
    lane = jax.lax.broadcasted_iota(jnp.int32, logits.shape, 1)
    logits = jnp.where(lane < C, logits, -1e30)
    m = jnp.max(logits, axis=-1, keepdims=True)
    e = jnp.exp(logits - m)
    s = jnp.sum(e, axis=-1, keepdims=True)
    out_ref[...] = (e / s)[:, :C]


@jax.jit
def kernel(x, a, W1, b1, W2, b2):
    n = a.shape[0]
    nr = n // RB

    xw = pl.pallas_call(
        _xw_kernel,
        grid=(n // 1000,),
        in_specs=[
            pl.BlockSpec((1000, D), lambda i: (i, 0)),
            pl.BlockSpec((D, H), lambda i: (0, 0)),
        ],
        out_specs=pl.BlockSpec((1000, H), lambda i: (i, 0)),
        out_shape=jax.ShapeDtypeStruct((n, H), jnp.float32),
    )(x, W1)

    w2p = jnp.zeros((H, CP), jnp.float32).at[:, :C].set(W2)
    b1r = b1.reshape(1, H)
    b2p = jnp.zeros((1, CP), jnp.float32).at[0, :C].set(b2)

    hw = pl.pallas_call(
        _layer1_kernel,
        grid=(nr,),
        in_specs=[
            pl.BlockSpec((RB, n), lambda i: (i, 0)),
            pl.BlockSpec((n, H), lambda i: (0, 0)),
            pl.BlockSpec((1, H), lambda i: (0, 0)),
            pl.BlockSpec((H, CP), lambda i: (0, 0)),
        ],
        out_specs=pl.BlockSpec((RB, CP), lambda i: (i, 0)),
        out_shape=jax.ShapeDtypeStruct((n, CP), jnp.float32),
        compiler_params=pltpu.CompilerParams(
            dimension_semantics=("arbitrary",)),
    )(a, xw, b1r, w2p)

    out = pl.pallas_call(
        _layer2_kernel,
        grid=(nr,),
        in_specs=[
            pl.BlockSpec((RB, n), lambda i: (i, 0)),
            pl.BlockSpec((n, CP), lambda i: (0, 0)),
            pl.BlockSpec((1, CP), lambda i: (0, 0)),
        ],
        out_specs=pl.BlockSpec((RB, C), lambda i: (i, 0)),
        out_shape=jax.ShapeDtypeStruct((n, C), jnp.float32),
        compiler_params=pltpu.CompilerParams(
            dimension_semantics=("arbitrary",)),
    )(a, hw, b2p)

    return out
